# baseline (speedup 1.0000x reference)
import functools

import jax
import jax.numpy as jnp
from jax import lax
from jax.experimental import pallas as pl
from jax.experimental.pallas import tpu as pltpu
from jax.experimental.pallas import tpu_sc as plsc

N = 62
NP = 64
WP = 80
NTRIL = N * (N + 1) // 2
ZSLOT = NTRIL
CBASE = 1968
C2B_SLOT = 2032
FCB_SLOT = 2040
STAGE = 2048
H = 128


def _sc_build(ew_hbm, c_hbm, c2b_hbm, fcb_hbm, linw_hbm, fcw_hbm, linb_hbm,
              wdp_hbm, par_hbm, stage_v, wd_v):
    wid = lax.axis_index("s") + lax.axis_index("c")

    @pl.when(wid == 0)
    def _():
        zeros = jnp.zeros((16,), jnp.float32)
        stage_v[pl.ds(NTRIL - 1, 16)] = zeros
        stage_v[pl.ds(2016, 16)] = zeros
        stage_v[pl.ds(2032, 16)] = zeros
        pltpu.sync_copy(ew_hbm, stage_v.at[pl.ds(0, NTRIL)])
        pltpu.sync_copy(c_hbm, stage_v.at[pl.ds(CBASE, N)])
        pltpu.sync_copy(c2b_hbm, stage_v.at[pl.ds(C2B_SLOT, 1)])
        pltpu.sync_copy(fcb_hbm, stage_v.at[pl.ds(FCB_SLOT, 3)])
        lanes = lax.iota(jnp.int32, 16)
        for c in range(NP // 16):
            j = lanes + (16 * c)
            in_bounds = j < N
            for r in range(N):
                a = jnp.maximum(j, r)
                b = jnp.minimum(j, r)
                t = lax.shift_right_logical(a * (a + 1), 1) + b
                idx = jnp.where(in_bounds, t, ZSLOT)
                wd_v[r, pl.ds(c * 16, 16)] = plsc.load_gather(stage_v, [idx])
        for r in range(N):
            idx = jnp.where(lanes == 0, CBASE + r, ZSLOT)
            wd_v[r, pl.ds(NP, 16)] = plsc.load_gather(stage_v, [idx])
        idx = jnp.where(lanes == 0, C2B_SLOT,
                        jnp.where(lanes < 4, FCB_SLOT - 1 + lanes, ZSLOT))
        wd_v[N, pl.ds(0, 16)] = plsc.load_gather(stage_v, [idx])
        zidx = jnp.full((16,), ZSLOT, jnp.int32)
        for c in range(1, WP // 16):
            wd_v[N, pl.ds(c * 16, 16)] = plsc.load_gather(stage_v, [zidx])
        pltpu.sync_copy(wd_v, wdp_hbm)
        pltpu.sync_copy(linw_hbm, par_hbm.at[pl.ds(0, H)])
        pltpu.sync_copy(fcw_hbm, par_hbm.at[pl.ds(H, 3)])
        pltpu.sync_copy(linb_hbm, par_hbm.at[pl.ds(H + 3, 1)])


@functools.cache
def _build_packed():
    return pl.kernel(
        _sc_build,
        out_type=(jax.ShapeDtypeStruct((N + 1, WP), jnp.float32),
                  jax.ShapeDtypeStruct((H + 4, H), jnp.float32)),
        mesh=plsc.VectorSubcoreMesh(core_axis_name="c", subcore_axis_name="s",
                                    num_cores=1, num_subcores=1),
        scratch_types=[
            pltpu.VMEM((STAGE,), jnp.float32),
            pltpu.VMEM((N + 1, WP), jnp.float32),
        ],
        compiler_params=pltpu.CompilerParams(needs_layout_passes=False),
    )


def _tc_body(wp_ref, pp_ref, x_ref, out_ref):
    Wd = wp_ref[0:N, 0:NP]
    cv = wp_ref[0:N, NP:NP + 1]
    c2b = wp_ref[N, 0]
    fcb = wp_ref[N:N + 1, 1:4]
    linw = pp_ref[0:H, :]
    fcw = pp_ref[H:H + 3, :]
    linb = pp_ref[H + 3:H + 4, :]
    absW = jnp.abs(Wd)
    deg_c = jnp.sum(absW, axis=1, keepdims=True)
    deg_r = jnp.sum(absW, axis=0, keepdims=True)
    dis_c = jnp.where(deg_c > 0,
                      lax.rsqrt(jnp.where(deg_c > 0, deg_c, 1.0)), 0.0)
    dis_r = jnp.where(deg_r > 0,
                      lax.rsqrt(jnp.where(deg_r > 0, deg_r, 1.0)), 0.0)
    A = Wd * dis_c * dis_r
    u_row = jnp.sum(A * cv, axis=0, keepdims=True)
    w = jnp.sum(A * u_row, axis=1, keepdims=True)
    X = x_ref[...]
    V = jnp.sum(X * w[None, :, :], axis=1)
    bias = jnp.sum(cv) * linb + c2b
    Y = lax.dot_general(V, linw, (((1,), (1,)), ((), ())),
                        preferred_element_type=jnp.float32,
                        precision=lax.Precision.HIGHEST) + bias
    Y = jnp.maximum(Y, 0.0)
    out_ref[...] = lax.dot_general(Y, fcw, (((1,), (1,)), ((), ())),
                                   preferred_element_type=jnp.float32,
                                   precision=lax.Precision.HIGHEST) + fcb


def kernel(X, ew, lin_W, lin_b, conv2_w, conv2_b, fc_W, fc_b, edge_index):
    del edge_index
    wd_plus, params = _build_packed()(ew, conv2_w.reshape(N), conv2_b, fc_b,
                                      lin_W, fc_W, lin_b.reshape(1, H))
    out = pl.pallas_call(
        _tc_body,
        out_shape=jax.ShapeDtypeStruct((X.shape[0], 3), jnp.float32),
    )(wd_plus, params, X)
    return out


# speedup vs baseline: 1.2309x; 1.2309x over previous
import functools

import jax
import jax.numpy as jnp
from jax import lax
from jax.experimental import pallas as pl
from jax.experimental.pallas import tpu as pltpu
from jax.experimental.pallas import tpu_sc as plsc

N = 62
NP = 64
NTRIL = N * (N + 1) // 2
ZSLOT = NTRIL
EW_PAD = 1968


def _sc_build_wd(ew_hbm, wd_hbm, ew_v, wd_v):
    wid = lax.axis_index("s") + lax.axis_index("c")

    @pl.when(wid == 0)
    def _():
        ew_v[pl.ds(EW_PAD - 16, 16)] = jnp.zeros((16,), jnp.float32)
        pltpu.sync_copy(ew_hbm, ew_v.at[pl.ds(0, NTRIL)])
        lanes = lax.iota(jnp.int32, 16)
        for c in range(NP // 16):
            j = lanes + (16 * c)
            in_bounds = j < N
            for r in range(N):
                a = jnp.maximum(j, r)
                b = jnp.minimum(j, r)
                t = lax.shift_right_logical(a * (a + 1), 1) + b
                idx = jnp.where(in_bounds, t, ZSLOT)
                wd_v[r, pl.ds(c * 16, 16)] = plsc.load_gather(ew_v, [idx])
        pltpu.sync_copy(wd_v, wd_hbm)


@functools.cache
def _build_wd():
    return pl.kernel(
        _sc_build_wd,
        out_type=jax.ShapeDtypeStruct((N, NP), jnp.float32),
        mesh=plsc.VectorSubcoreMesh(core_axis_name="c", subcore_axis_name="s",
                                    num_cores=1, num_subcores=1),
        scratch_types=[
            pltpu.VMEM((EW_PAD,), jnp.float32),
            pltpu.VMEM((N, NP), jnp.float32),
        ],
        compiler_params=pltpu.CompilerParams(needs_layout_passes=False),
    )


def _tc_body(wd_ref, x_ref, c_ref, linw_ref, linb_ref, c2b_ref, fcw_ref,
             fcb_ref, out_ref):
    Wd = wd_ref[...]
    absW = jnp.abs(Wd)
    deg_c = jnp.sum(absW, axis=1, keepdims=True)
    deg_r = jnp.sum(absW, axis=0, keepdims=True)
    dis_c = jnp.where(deg_c > 0,
                      lax.rsqrt(jnp.where(deg_c > 0, deg_c, 1.0)), 0.0)
    dis_r = jnp.where(deg_r > 0,
                      lax.rsqrt(jnp.where(deg_r > 0, deg_r, 1.0)), 0.0)
    A = Wd * dis_c * dis_r
    cv = c_ref[0, :, :]
    u_row = jnp.sum(A * cv, axis=0, keepdims=True)
    w = jnp.sum(A * u_row, axis=1, keepdims=True)
    X = x_ref[...]
    V = jnp.sum(X * w[None, :, :], axis=1)
    linb = linb_ref[...].reshape(1, -1)
    bias = jnp.sum(cv) * linb + c2b_ref[0]
    Y = lax.dot_general(V, linw_ref[...], (((1,), (1,)), ((), ())),
                        preferred_element_type=jnp.float32,
                        precision=lax.Precision.HIGHEST) + bias
    Y = jnp.maximum(Y, 0.0)
    out_ref[...] = lax.dot_general(Y, fcw_ref[...], (((1,), (1,)), ((), ())),
                                   preferred_element_type=jnp.float32,
                                   precision=lax.Precision.HIGHEST) \
        + fcb_ref[...].reshape(1, -1)


def kernel(X, ew, lin_W, lin_b, conv2_w, conv2_b, fc_W, fc_b, edge_index):
    del edge_index
    wd = _build_wd()(ew)
    out = pl.pallas_call(
        _tc_body,
        out_shape=jax.ShapeDtypeStruct((X.shape[0], fc_W.shape[0]),
                                       jnp.float32),
    )(wd, X, conv2_w, lin_W, lin_b, conv2_b, fc_W, fc_b)
    return out
